# baseline (speedup 1.0000x reference)
import functools

import jax
import jax.numpy as jnp
from jax import lax
from jax.experimental import pallas as pl
from jax.experimental.pallas import tpu as pltpu
from jax.experimental.pallas import tpu_sc as plsc


def _segpe_table(seg_table, pe2d):
    seg_n, d = seg_table.shape
    l_n = pe2d.shape[0]

    def body(seg_ref, pe_ref, out_ref):
        out_ref[...] = seg_ref[...][:, None, :] + pe_ref[...][None, :, :]

    return pl.pallas_call(
        body,
        out_shape=jax.ShapeDtypeStruct((seg_n, l_n, d), jnp.float32),
    )(seg_table, pe2d)


def _make_sc_gather(n_rows, d, l_n, seg_n, chunk):
    info = plsc.get_sparse_core_info()
    nw = info.num_cores * info.num_subcores
    lanes = info.num_lanes
    assert n_rows % (nw * chunk) == 0
    per_w = n_rows // nw
    n_chunks = per_w // chunk
    assert n_chunks % 2 == 0 and chunk <= 128
    mesh = plsc.VectorSubcoreMesh(core_axis_name="c", subcore_axis_name="s")

    @functools.partial(
        pl.kernel,
        mesh=mesh,
        out_type=jax.ShapeDtypeStruct((n_rows, d), jnp.float32),
        scratch_types=[
            pltpu.VMEM((n_chunks, chunk), jnp.int32),
            pltpu.VMEM((n_chunks, chunk), jnp.int32),
            pltpu.VMEM((seg_n * l_n, d), jnp.float32),
            pltpu.VMEM((chunk, d), jnp.float32),
            pltpu.VMEM((chunk, d), jnp.float32),
            pltpu.SemaphoreType.DMA,
            pltpu.SemaphoreType.DMA,
            pltpu.SemaphoreType.DMA,
            pltpu.SemaphoreType.DMA,
        ],
    )
    def sc_kernel(seq_hbm, seg_hbm, tok_hbm, segpe_hbm, out_hbm,
                  seqi_v, idx2_v, spe_v, tok_a, tok_b,
                  st_a, st_b, so_a, so_b):
        wid = lax.axis_index("s") * info.num_cores + lax.axis_index("c")
        base = wid * per_w
        tok_bufs = (tok_a, tok_b)
        sems_t, sems_o = (st_a, st_b), (so_a, so_b)

        pltpu.sync_copy(segpe_hbm, spe_v)
        pltpu.sync_copy(seq_hbm.at[wid], seqi_v)
        pltpu.sync_copy(seg_hbm.at[wid], idx2_v)

        def idx_body(i, carry):
            for j in range(chunk // lanes):
                sl = pl.ds(j * lanes, lanes)
                flat = (base + i * chunk + j * lanes) + lax.iota(jnp.int32, lanes)
                idx2_v[i, sl] = idx2_v[i, sl] * l_n + flat % l_n
            return carry

        lax.fori_loop(0, n_chunks, idx_body, 0)

        def tok_dma(g, b):
            return pltpu.make_async_copy(tok_hbm.at[seqi_v.at[g]], tok_bufs[b], sems_t[b])

        def out_dma(g, b):
            return pltpu.make_async_copy(
                tok_bufs[b], out_hbm.at[pl.ds(base + g * chunk, chunk)], sems_o[b])

        def iter_body(t, carry):
            for b in range(2):
                g = 2 * t + b

                @pl.when(g >= 2)
                def _drain_out():
                    out_dma(g - 2, b).wait()

                tok_dma(g, b).start()
                tok_dma(g, b).wait()

                def add_body(q, c2):
                    i2vec = idx2_v[g, pl.ds(q * lanes, lanes)]
                    for e in range(lanes):
                        i2s = i2vec[e]
                        r = q * lanes + e
                        for c in range(d // lanes):
                            sl = pl.ds(c * lanes, lanes)
                            tok_bufs[b][r, sl] = tok_bufs[b][r, sl] + spe_v[i2s, sl]
                    return c2

                lax.fori_loop(0, chunk // lanes, add_body, 0)
                out_dma(g, b).start()
            return carry

        lax.fori_loop(0, n_chunks // 2, iter_body, 0)
        out_dma(n_chunks - 2, 0).wait()
        out_dma(n_chunks - 1, 1).wait()

    return sc_kernel


def kernel(sequence, segment, token_table, seg_table, pe):
    b, l_n = sequence.shape
    d = token_table.shape[1]
    seg_n = seg_table.shape[0]
    n_rows = b * l_n
    chunk = 128

    nw = 32
    seq3d = sequence.reshape(nw, n_rows // (nw * chunk), chunk).astype(jnp.int32)
    seg3d = segment.reshape(nw, n_rows // (nw * chunk), chunk).astype(jnp.int32)
    segpe = _segpe_table(seg_table, pe[0, :l_n]).reshape(seg_n * l_n, d)

    sc = _make_sc_gather(n_rows, d, l_n, seg_n, chunk)
    out_flat = sc(seq3d, seg3d, token_table, segpe)
    return out_flat.reshape(b, l_n, d)


# speedup vs baseline: 2.0263x; 2.0263x over previous
import functools

import jax
import jax.numpy as jnp
from jax import lax
from jax.experimental import pallas as pl
from jax.experimental.pallas import tpu as pltpu
from jax.experimental.pallas import tpu_sc as plsc


def _segpe_table(seg_table, pe2d):
    seg_n, d = seg_table.shape
    l_n = pe2d.shape[0]

    def body(seg_ref, pe_ref, out_ref):
        out_ref[...] = seg_ref[...][:, None, :] + pe_ref[...][None, :, :]

    return pl.pallas_call(
        body,
        out_shape=jax.ShapeDtypeStruct((seg_n, l_n, d), jnp.float32),
    )(seg_table, pe2d)


def _make_sc_gather(n_rows, d, l_n, seg_n, chunk):
    info = plsc.get_sparse_core_info()
    nw = info.num_cores * info.num_subcores
    lanes = info.num_lanes
    assert n_rows % (nw * chunk) == 0
    per_w = n_rows // nw
    n_chunks = per_w // chunk
    assert n_chunks % 2 == 0 and chunk <= 128
    mesh = plsc.VectorSubcoreMesh(core_axis_name="c", subcore_axis_name="s")

    @functools.partial(
        pl.kernel,
        mesh=mesh,
        out_type=jax.ShapeDtypeStruct((n_rows, d), jnp.float32),
        scratch_types=[
            pltpu.VMEM((n_chunks, chunk), jnp.int32),
            pltpu.VMEM((n_chunks, chunk), jnp.int32),
            pltpu.VMEM_SHARED((seg_n * l_n, d), jnp.float32),
            pltpu.VMEM((chunk, d), jnp.float32),
            pltpu.VMEM((chunk, d), jnp.float32),
            pltpu.VMEM((chunk, d), jnp.float32),
            pltpu.VMEM((chunk, d), jnp.float32),
            pltpu.VMEM((chunk, d), jnp.float32),
            pltpu.VMEM((chunk, d), jnp.float32),
            pltpu.SemaphoreType.DMA,
            pltpu.SemaphoreType.DMA,
            pltpu.SemaphoreType.DMA,
            pltpu.SemaphoreType.DMA,
            pltpu.SemaphoreType.DMA,
            pltpu.SemaphoreType.DMA,
        ],
    )
    def sc_kernel(seq_hbm, seg_hbm, tok_hbm, segpe_hbm, out_hbm,
                  seqi_v, idx2_v, spe_sh, tok_a, tok_b, spe_a, spe_b,
                  oub_a, oub_b, st_a, st_b, ss_a, ss_b, so_a, so_b):
        wid = lax.axis_index("s") * info.num_cores + lax.axis_index("c")
        base = wid * per_w
        tok_bufs, spe_bufs, out_bufs = (tok_a, tok_b), (spe_a, spe_b), (oub_a, oub_b)
        sems_t, sems_s, sems_o = (st_a, st_b), (ss_a, ss_b), (so_a, so_b)

        @pl.when(lax.axis_index("s") == 0)
        def _stage():
            pltpu.sync_copy(segpe_hbm, spe_sh)

        pltpu.sync_copy(seq_hbm.at[wid], seqi_v)
        pltpu.sync_copy(seg_hbm.at[wid], idx2_v)

        def idx_body(i, carry):
            for j in range(chunk // lanes):
                sl = pl.ds(j * lanes, lanes)
                flat = (base + i * chunk + j * lanes) + lax.iota(jnp.int32, lanes)
                idx2_v[i, sl] = idx2_v[i, sl] * l_n + flat % l_n
            return carry

        lax.fori_loop(0, n_chunks, idx_body, 0)
        plsc.subcore_barrier()

        def tok_dma(g, b):
            return pltpu.make_async_copy(tok_hbm.at[seqi_v.at[g]], tok_bufs[b], sems_t[b])

        def spe_dma(g, b):
            return pltpu.make_async_copy(spe_sh.at[idx2_v.at[g]], spe_bufs[b], sems_s[b])

        def out_dma(g, b):
            return pltpu.make_async_copy(
                out_bufs[b], out_hbm.at[pl.ds(base + g * chunk, chunk)], sems_o[b])

        def iter_body(t, carry):
            for b in range(2):
                g = 2 * t + b
                tok_dma(g, b).start()
                spe_dma(g, b).start()
                tok_dma(g, b).wait()
                spe_dma(g, b).wait()

                @pl.when(g >= 2)
                def _drain_out():
                    out_dma(g - 2, b).wait()

                def add_body(r, c2):
                    for c in range(d // lanes):
                        sl = pl.ds(c * lanes, lanes)
                        out_bufs[b][r, sl] = tok_bufs[b][r, sl] + spe_bufs[b][r, sl]
                    return c2

                lax.fori_loop(0, chunk, add_body, 0)
                out_dma(g, b).start()
            return carry

        lax.fori_loop(0, n_chunks // 2, iter_body, 0)
        out_dma(n_chunks - 2, 0).wait()
        out_dma(n_chunks - 1, 1).wait()

    return sc_kernel


def kernel(sequence, segment, token_table, seg_table, pe):
    b, l_n = sequence.shape
    d = token_table.shape[1]
    seg_n = seg_table.shape[0]
    n_rows = b * l_n
    chunk = 128

    nw = 32
    seq3d = sequence.reshape(nw, n_rows // (nw * chunk), chunk).astype(jnp.int32)
    seg3d = segment.reshape(nw, n_rows // (nw * chunk), chunk).astype(jnp.int32)
    segpe = _segpe_table(seg_table, pe[0, :l_n]).reshape(seg_n * l_n, d)

    sc = _make_sc_gather(n_rows, d, l_n, seg_n, chunk)
    out_flat = sc(seq3d, seg3d, token_table, segpe)
    return out_flat.reshape(b, l_n, d)


# speedup vs baseline: 3.2659x; 1.6118x over previous
import functools

import jax
import jax.numpy as jnp
from jax import lax
from jax.experimental import pallas as pl
from jax.experimental.pallas import tpu as pltpu
from jax.experimental.pallas import tpu_sc as plsc


def _segpe_table(seg_table, pe2d):
    seg_n, d = seg_table.shape
    l_n = pe2d.shape[0]

    def body(seg_ref, pe_ref, out_ref):
        out_ref[...] = seg_ref[...][:, None, :] + pe_ref[...][None, :, :]

    return pl.pallas_call(
        body,
        out_shape=jax.ShapeDtypeStruct((seg_n, l_n, d), jnp.float32),
    )(seg_table, pe2d)


def _make_sc_gather(n_rows, d, l_n, seg_n, chunk):
    info = plsc.get_sparse_core_info()
    nw = info.num_cores * info.num_subcores
    lanes = info.num_lanes
    assert n_rows % (nw * chunk) == 0
    per_w = n_rows // nw
    n_chunks = per_w // chunk
    assert n_chunks % 2 == 0 and chunk <= 128
    mesh = plsc.VectorSubcoreMesh(core_axis_name="c", subcore_axis_name="s")

    @functools.partial(
        pl.kernel,
        mesh=mesh,
        out_type=jax.ShapeDtypeStruct((n_rows, d), jnp.float32),
        scratch_types=[
            pltpu.VMEM((n_chunks, chunk), jnp.int32),
            pltpu.VMEM((n_chunks, chunk), jnp.int32),
            pltpu.VMEM_SHARED((seg_n * l_n, d), jnp.float32),
            pltpu.VMEM((chunk, d), jnp.float32),
            pltpu.VMEM((chunk, d), jnp.float32),
            pltpu.VMEM((chunk, d), jnp.float32),
            pltpu.VMEM((chunk, d), jnp.float32),
            pltpu.VMEM((chunk, d), jnp.float32),
            pltpu.VMEM((chunk, d), jnp.float32),
            pltpu.SemaphoreType.DMA,
            pltpu.SemaphoreType.DMA,
            pltpu.SemaphoreType.DMA,
            pltpu.SemaphoreType.DMA,
            pltpu.SemaphoreType.DMA,
            pltpu.SemaphoreType.DMA,
        ],
    )
    def sc_kernel(seq_hbm, seg_hbm, tok_hbm, segpe_hbm, out_hbm,
                  seqi_v, idx2_v, spe_sh, tok_a, tok_b, spe_a, spe_b,
                  oub_a, oub_b, st_a, st_b, ss_a, ss_b, so_a, so_b):
        wid = lax.axis_index("s") * info.num_cores + lax.axis_index("c")
        base = wid * per_w
        tok_bufs, spe_bufs, out_bufs = (tok_a, tok_b), (spe_a, spe_b), (oub_a, oub_b)
        sems_t, sems_s, sems_o = (st_a, st_b), (ss_a, ss_b), (so_a, so_b)

        @pl.when(lax.axis_index("s") == 0)
        def _stage():
            pltpu.sync_copy(segpe_hbm, spe_sh)

        pltpu.sync_copy(seq_hbm.at[wid], seqi_v)
        pltpu.sync_copy(seg_hbm.at[wid], idx2_v)

        def idx_body(i, carry):
            for j in range(chunk // lanes):
                sl = pl.ds(j * lanes, lanes)
                flat = (base + i * chunk + j * lanes) + lax.iota(jnp.int32, lanes)
                idx2_v[i, sl] = idx2_v[i, sl] * l_n + flat % l_n
            return carry

        lax.fori_loop(0, n_chunks, idx_body, 0)
        plsc.subcore_barrier()

        def tok_dma(g, b):
            return pltpu.make_async_copy(tok_hbm.at[seqi_v.at[g]], tok_bufs[b], sems_t[b])

        def spe_dma(g, b):
            return pltpu.make_async_copy(spe_sh.at[idx2_v.at[g]], spe_bufs[b], sems_s[b])

        def out_dma(g, b):
            return pltpu.make_async_copy(
                out_bufs[b], out_hbm.at[pl.ds(base + g * chunk, chunk)], sems_o[b])

        tok_dma(0, 0).start()
        spe_dma(0, 0).start()

        def iter_body(t, carry):
            for b in range(2):
                g = 2 * t + b

                @pl.when(g + 1 < n_chunks)
                def _prefetch():
                    tok_dma(g + 1, 1 - b).start()
                    spe_dma(g + 1, 1 - b).start()

                tok_dma(g, b).wait()
                spe_dma(g, b).wait()

                @pl.when(g >= 2)
                def _drain_out():
                    out_dma(g - 2, b).wait()

                def add_body(r, c2):
                    for c in range(d // lanes):
                        sl = pl.ds(c * lanes, lanes)
                        out_bufs[b][r, sl] = tok_bufs[b][r, sl] + spe_bufs[b][r, sl]
                    return c2

                lax.fori_loop(0, chunk, add_body, 0)
                out_dma(g, b).start()
            return carry

        lax.fori_loop(0, n_chunks // 2, iter_body, 0)
        out_dma(n_chunks - 2, 0).wait()
        out_dma(n_chunks - 1, 1).wait()

    return sc_kernel


def kernel(sequence, segment, token_table, seg_table, pe):
    b, l_n = sequence.shape
    d = token_table.shape[1]
    seg_n = seg_table.shape[0]
    n_rows = b * l_n
    chunk = 128

    nw = 32
    seq3d = sequence.reshape(nw, n_rows // (nw * chunk), chunk).astype(jnp.int32)
    seg3d = segment.reshape(nw, n_rows // (nw * chunk), chunk).astype(jnp.int32)
    segpe = _segpe_table(seg_table, pe[0, :l_n]).reshape(seg_n * l_n, d)

    sc = _make_sc_gather(n_rows, d, l_n, seg_n, chunk)
    out_flat = sc(seq3d, seg3d, token_table, segpe)
    return out_flat.reshape(b, l_n, d)
